# baseline (speedup 1.0000x reference)
import functools

import jax
import jax.numpy as jnp
from jax import lax
from jax.experimental import pallas as pl
from jax.experimental.pallas import tpu as pltpu
from jax.experimental.pallas import tpu_sc as plsc

NUM_CORES = 2
NUM_SUBCORES = 16
NUM_WORKERS = NUM_CORES * NUM_SUBCORES
LANES = 16

BATCH = 16384
EMBED_DIM = 32
ROWS_PER_TILE = 8
B_PER_W = BATCH // NUM_WORKERS
SUB = 32
N_SUB = B_PER_W // SUB
N_BLOCKS = SUB // LANES


def _mf_body(uid_hbm, iid_hbm, utab_hbm, itab_hbm, bias_hbm, out_hbm,
             uidx, iidx, utiles, itiles, btile, out_v, sem, bsem):
    wid = lax.axis_index("s") * NUM_CORES + lax.axis_index("c")
    base = wid * B_PER_W

    pltpu.sync_copy(uid_hbm.at[pl.ds(base, B_PER_W)], uidx)
    pltpu.sync_copy(iid_hbm.at[pl.ds(base, B_PER_W)], iidx)

    @pl.loop(0, N_SUB, unroll=1)
    def _sub_batch(sb):
        k0 = pl.multiple_of(sb * SUB, SUB)
        copies = []
        for bi in range(N_BLOCKS):
            b0 = pl.multiple_of(k0 + bi * LANES, LANES)
            uv = uidx[pl.ds(b0, LANES)]
            iv = iidx[pl.ds(b0, LANES)]
            for j in range(LANES):
                k = bi * LANES + j
                ur = pl.multiple_of(uv[j] & -8, ROWS_PER_TILE)
                ir = pl.multiple_of(iv[j] & -8, ROWS_PER_TILE)
                copies.append(pltpu.async_copy(
                    utab_hbm.at[pl.ds(ur, ROWS_PER_TILE)], utiles.at[k],
                    sem))
                copies.append(pltpu.async_copy(
                    itab_hbm.at[pl.ds(ir, ROWS_PER_TILE)], itiles.at[k],
                    sem))
                copies.append(pltpu.async_copy(
                    bias_hbm.at[pl.ds(ir, ROWS_PER_TILE)],
                    btile.at[k], bsem))
        for cp in copies:
            cp.wait()
        for bi in range(N_BLOCKS):
            b0 = pl.multiple_of(k0 + bi * LANES, LANES)
            klocal = bi * LANES + lax.iota(jnp.int32, LANES)
            usub = uidx[pl.ds(b0, LANES)] & 7
            isub = iidx[pl.ds(b0, LANES)] & 7
            acc = plsc.load_gather(btile, [klocal, isub,
                                           jnp.zeros((LANES,), jnp.int32)])
            for d in range(EMBED_DIM):
                ds_ = jnp.full((LANES,), d, jnp.int32)
                acc = acc + (
                    plsc.load_gather(utiles, [klocal, usub, ds_])
                    * plsc.load_gather(itiles, [klocal, isub, ds_]))
            out_v[pl.ds(b0, LANES)] = acc

    pltpu.sync_copy(out_v, out_hbm.at[pl.ds(base, B_PER_W)])


_mf_call = functools.partial(
    pl.kernel,
    out_type=jax.ShapeDtypeStruct((BATCH,), jnp.float32),
    mesh=plsc.VectorSubcoreMesh(core_axis_name="c", subcore_axis_name="s",
                                num_cores=NUM_CORES,
                                num_subcores=NUM_SUBCORES),
    scratch_types=[
        pltpu.VMEM((B_PER_W,), jnp.int32),
        pltpu.VMEM((B_PER_W,), jnp.int32),
        pltpu.VMEM((SUB, ROWS_PER_TILE, EMBED_DIM), jnp.float32),
        pltpu.VMEM((SUB, ROWS_PER_TILE, EMBED_DIM), jnp.float32),
        pltpu.VMEM((SUB, ROWS_PER_TILE, 1), jnp.float32),
        pltpu.VMEM((B_PER_W,), jnp.float32),
        pltpu.SemaphoreType.DMA,
        pltpu.SemaphoreType.DMA,
    ],
    compiler_params=pltpu.CompilerParams(needs_layout_passes=False,
                                         use_tc_tiling_on_sc=True),
)(_mf_body)


@jax.jit
def kernel(user_ids, item_ids, user_table, item_table, item_bias_table):
    return _mf_call(user_ids.astype(jnp.int32), item_ids.astype(jnp.int32),
                    user_table, item_table, item_bias_table)
